# baseline (speedup 1.0000x reference)
import jax
import jax.numpy as jnp
from jax.experimental import pallas as pl

_N = 100000
_OBS_D = 17
_ACT_D = 6
_BC = 100096


def _copy_body(obs_ref, act_ref, obs_out, act_out):
    obs_out[...] = obs_ref[...]
    act_out[...] = act_ref[...]


def kernel(obs_table, act_table):
    obs_t = obs_table.T
    act_t = act_table.T

    grid = (_N + _BC - 1) // _BC
    obs_o, act_o = pl.pallas_call(
        _copy_body,
        grid=(grid,),
        in_specs=[
            pl.BlockSpec((_OBS_D, _BC), lambda i: (0, i)),
            pl.BlockSpec((_ACT_D, _BC), lambda i: (0, i)),
        ],
        out_specs=[
            pl.BlockSpec((_OBS_D, _BC), lambda i: (0, i)),
            pl.BlockSpec((_ACT_D, _BC), lambda i: (0, i)),
        ],
        out_shape=[
            jax.ShapeDtypeStruct((_OBS_D, _N), jnp.float32),
            jax.ShapeDtypeStruct((_ACT_D, _N), jnp.float32),
        ],
    )(obs_t, act_t)

    return (obs_o.T, act_o.T)


# speedup vs baseline: 1.1563x; 1.1563x over previous
import jax
import jax.numpy as jnp
from jax.experimental import pallas as pl

_N = 100000
_OBS_D = 17
_ACT_D = 6
_BC = 50048


def _copy_body(obs_ref, act_ref, obs_out, act_out):
    obs_out[...] = obs_ref[...]
    act_out[...] = act_ref[...]


def kernel(obs_table, act_table):
    obs_t = obs_table.T
    act_t = act_table.T

    grid = (_N + _BC - 1) // _BC
    obs_o, act_o = pl.pallas_call(
        _copy_body,
        grid=(grid,),
        in_specs=[
            pl.BlockSpec((_OBS_D, _BC), lambda i: (0, i)),
            pl.BlockSpec((_ACT_D, _BC), lambda i: (0, i)),
        ],
        out_specs=[
            pl.BlockSpec((_OBS_D, _BC), lambda i: (0, i)),
            pl.BlockSpec((_ACT_D, _BC), lambda i: (0, i)),
        ],
        out_shape=[
            jax.ShapeDtypeStruct((_OBS_D, _N), jnp.float32),
            jax.ShapeDtypeStruct((_ACT_D, _N), jnp.float32),
        ],
    )(obs_t, act_t)

    return (obs_o.T, act_o.T)
